# Initial kernel scaffold; baseline (speedup 1.0000x reference)
import jax
import jax.numpy as jnp
from jax.experimental import pallas as pl


def kernel(target, sub_graph_nodes, budget, feat, nor_adj_tensor, node_emb, wlabel, wsec, train_flag, weight1, weight2, a_w1, a_b1, a_w2, a_b2, a_w3, a_b3, e_w1, e_b1, e_w2, e_b2, e_w3, e_b3):
    raise NotImplementedError("write your pallas kernel here")


# speedup vs baseline: 2.1199x; 2.1199x over previous
import functools

import jax
import jax.numpy as jnp
from jax import lax
from jax.experimental import pallas as pl
from jax.experimental.pallas import tpu as pltpu
from jax.experimental.pallas import tpu_sc as plsc

N = 100000
S = 50000
D = 128
NW = 32
CHUNK = 128
CPW = 13
SP = NW * CPW * CHUNK
PAD = SP - S
NROW = SP // 128
BR = 512
INT_MIN = -2147483648


def _sc_gather_body(idx_hbm, feat_hbm, emb_hbm, g_hbm, psum_hbm,
                    idx_v, fbuf, ebuf, acc_v, fsem, esem):
    wid = lax.axis_index("s") * 2 + lax.axis_index("c")
    pltpu.sync_copy(idx_hbm.at[wid], idx_v)

    def chunk_body(c, acc):
        fcopy = pltpu.async_copy(feat_hbm.at[idx_v.at[c]], fbuf, fsem)
        ecopy = pltpu.async_copy(emb_hbm.at[idx_v.at[c]], ebuf, esem)
        fcopy.wait()
        pltpu.sync_copy(fbuf, g_hbm.at[wid * CPW + c])
        ecopy.wait()

        def row_body(i, a):
            return tuple(a[j] + ebuf[i, pl.ds(j * 16, 16)] for j in range(8))

        return lax.fori_loop(0, CHUNK, row_body, acc)

    acc0 = tuple(jnp.zeros((16,), jnp.float32) for _ in range(8))
    acc = lax.fori_loop(0, CPW, chunk_body, acc0)
    for j in range(8):
        acc_v[pl.ds(j * 16, 16)] = acc[j]
    pltpu.sync_copy(acc_v, psum_hbm.at[wid])


_sc_gather = functools.partial(
    pl.kernel,
    out_type=[
        jax.ShapeDtypeStruct((NW * CPW, CHUNK, D), jnp.float32),
        jax.ShapeDtypeStruct((NW, D), jnp.float32),
    ],
    mesh=plsc.VectorSubcoreMesh(core_axis_name="c", subcore_axis_name="s"),
    scratch_types=[
        pltpu.VMEM((CPW, CHUNK), jnp.int32),
        pltpu.VMEM((CHUNK, D), jnp.float32),
        pltpu.VMEM((CHUNK, D), jnp.float32),
        pltpu.VMEM((D,), jnp.float32),
        pltpu.SemaphoreType.DMA,
        pltpu.SemaphoreType.DMA,
    ],
)(_sc_gather_body)


def _leaky(x):
    return jnp.where(x >= 0, x, x * jnp.float32(0.01))


def _prep_body(tgt, psum, emb0, feat_t, emb_t, w1, w2, a_w1, a_b1, a_w2t,
               a_b2, a_w3t, a_b3, e_sub_t, e_tar_t, e_add_t, e_wl_t, e_ws_t,
               e_b1, wl, ws, add_feat_o, m_o, c1_o):
    f32 = jnp.float32
    dot = functools.partial(jnp.dot, preferred_element_type=f32)
    sub_emb = (jnp.sum(psum[...], axis=0, keepdims=True)
               - f32(PAD) * emb0[0]) * f32(1.0 / S)
    ft = feat_t[0]
    tmp = jnp.maximum(dot(ft, w1[...]), 0.0)
    tarfeat = dot(tmp, w2[...])
    aw1 = a_w1[...]
    h = a_b1[...]
    h = h + dot(sub_emb, aw1[:, 0:128].T)
    h = h + dot(emb_t[0], aw1[:, 128:256].T)
    h = h + dot(tarfeat, aw1[:, 256:384].T)
    h = h + dot(wl[...], aw1[:, 384:512].T)
    h = h + dot(ws[...], aw1[:, 512:640].T)
    h = _leaky(h)
    h = _leaky(dot(h, a_w2t[...]) + a_b2[...])
    add_feat = dot(h, a_w3t[...]) + a_b3[...]
    add_feat_o[...] = add_feat
    inj = jax.nn.sigmoid(add_feat)
    w12 = dot(w1[...], w2[...])
    tar_xw = dot(ft, w12)
    add_xw = dot(inj, w12)
    m_o[...] = dot(w12, e_sub_t[...])
    c1_o[...] = (e_b1[...] + dot(tar_xw, e_tar_t[...])
                 + dot(add_xw, e_add_t[...]) + dot(wl[...], e_wl_t[...])
                 + dot(ws[...], e_ws_t[...]))


def _prep_call(tgt, psum, node_emb, feat, w1, w2, a_w1, a_b1, a_w2t, a_b2,
               a_w3t, a_b3, e_sub_t, e_tar_t, e_add_t, e_wl_t, e_ws_t,
               e_b1, wl, ws):
    node_emb3 = node_emb.reshape(N, 1, D)
    feat3 = feat.reshape(N, 1, D)
    whole = lambda shp: pl.BlockSpec(shp, lambda i, t: (0,) * len(shp))
    tgt_row = pl.BlockSpec((1, 1, D), lambda i, t: (t[0], 0, 0))
    grid_spec = pltpu.PrefetchScalarGridSpec(
        num_scalar_prefetch=1,
        grid=(1,),
        in_specs=[
            whole((NW, D)),
            pl.BlockSpec((1, 1, D), lambda i, t: (0, 0, 0)),
            tgt_row,
            tgt_row,
            whole((D, 64)),
            whole((64, D)),
            whole((D, 640)),
            whole((1, D)),
            whole((D, 512)),
            whole((1, 512)),
            whole((512, D)),
            whole((1, D)),
            whole((D, 512)),
            whole((D, 512)),
            whole((D, 512)),
            whole((D, 512)),
            whole((D, 512)),
            whole((1, 512)),
            whole((1, D)),
            whole((1, D)),
        ],
        out_specs=[
            whole((1, D)),
            whole((D, 512)),
            whole((1, 512)),
        ],
    )
    return pl.pallas_call(
        _prep_body,
        grid_spec=grid_spec,
        out_shape=[
            jax.ShapeDtypeStruct((1, D), jnp.float32),
            jax.ShapeDtypeStruct((D, 512), jnp.float32),
            jax.ShapeDtypeStruct((1, 512), jnp.float32),
        ],
    )(tgt, psum, node_emb3, feat3, node_emb3, w1, w2, a_w1, a_b1, a_w2t, a_b2,
      a_w3t, a_b3, e_sub_t, e_tar_t, e_add_t, e_wl_t, e_ws_t, e_b1, wl, ws)


def _mlp_body(g, adj, m, c1, ecol, ew2t, eb2, ew3t, eb3, out):
    f32 = jnp.float32
    dot = functools.partial(jnp.dot, preferred_element_type=f32)
    h1 = dot(g[...], m[...]) + adj[...] * ecol[...] + c1[...]
    h1 = _leaky(h1)
    h2 = _leaky(dot(h1, ew2t[...]) + eb2[...])
    out[...] = dot(h2, ew3t[...]) + eb3[...]


def _mlp_call(g2, adjp, m, c1, ecol, ew2t, eb2, ew3t, eb3):
    row = lambda shp: pl.BlockSpec(shp, lambda i: (i, 0))
    whole = lambda shp: pl.BlockSpec(shp, lambda i: (0,) * len(shp))
    return pl.pallas_call(
        _mlp_body,
        grid=(SP // BR,),
        in_specs=[
            row((BR, D)),
            row((BR, 1)),
            whole((D, 512)),
            whole((1, 512)),
            whole((1, 512)),
            whole((512, 32)),
            whole((1, 32)),
            whole((32, 1)),
            whole((1, 1)),
        ],
        out_specs=row((BR, 1)),
        out_shape=jax.ShapeDtypeStruct((SP, 1), jnp.float32),
    )(g2, adjp, m, c1, ecol, ew2t, eb2, ew3t, eb3)


def _select_body(xin, score):
    i32 = jnp.int32
    f32 = jnp.float32
    r = lax.broadcasted_iota(i32, (NROW, 128), 0)
    c = lax.broadcasted_iota(i32, (NROW, 128), 1)
    flat = r * 128 + c
    x = jnp.where(flat < S, xin[...], f32(-3e38))
    imin = jnp.int32(INT_MIN)
    bi = lax.bitcast_convert_type(x, i32)
    key = jnp.where(bi >= 0, bi, jnp.bitwise_xor(jnp.bitwise_not(bi), imin))

    def bit_body(i, cand):
        bit = lax.shift_left(jnp.int32(1), jnp.int32(31) - i)
        trial = jnp.bitwise_or(cand, bit)
        thr = jnp.bitwise_xor(trial, imin)
        cnt = jnp.sum((key >= thr).astype(i32))
        return jnp.where(cnt >= 128, trial, cand)

    cand = lax.fori_loop(0, 32, bit_body, jnp.int32(0))
    thr = jnp.bitwise_xor(cand, imin)
    gt = key > thr
    eq = key == thr
    need = jnp.float32(128) - jnp.sum(gt.astype(i32)).astype(f32)
    eqf = eq.astype(f32)
    rr = lax.broadcasted_iota(i32, (128, 128), 0)
    cc = lax.broadcasted_iota(i32, (128, 128), 1)
    tri = (rr < cc).astype(f32)
    in_row = jnp.dot(eqf, tri, preferred_element_type=f32)
    rowtot = jnp.sum(eqf, axis=1, keepdims=True)
    r2 = lax.broadcasted_iota(i32, (NROW, NROW), 0)
    c2 = lax.broadcasted_iota(i32, (NROW, NROW), 1)
    ltri = (c2 < r2).astype(f32)
    row_off = jnp.dot(ltri, rowtot, preferred_element_type=f32)
    rank = row_off + in_row
    sel = jnp.logical_or(gt, jnp.logical_and(eq, rank < need))
    score[...] = jnp.where(sel, f32(1.0), f32(0.0))


_select_call = pl.pallas_call(
    _select_body,
    out_shape=jax.ShapeDtypeStruct((NROW, 128), jnp.float32),
)


def kernel(target, sub_graph_nodes, budget, feat, nor_adj_tensor, node_emb,
           wlabel, wsec, train_flag, weight1, weight2, a_w1, a_b1, a_w2, a_b2,
           a_w3, a_b3, e_w1, e_b1, e_w2, e_b2, e_w3, e_b3):
    idx = sub_graph_nodes.astype(jnp.int32)
    idxp = jnp.concatenate([idx, jnp.zeros((PAD,), jnp.int32)])
    idxp = idxp.reshape(NW, CPW, CHUNK)

    g3, psum = _sc_gather(idxp, feat, node_emb)

    add_feat2, m, c1 = _prep_call(
        target.astype(jnp.int32).reshape(1), psum, node_emb, feat,
        weight1, weight2, a_w1, a_b1[None, :], a_w2.T, a_b2[None, :],
        a_w3.T, a_b3[None, :],
        e_w1[:, 128:256].T, e_w1[:, 0:128].T, e_w1[:, 256:384].T,
        e_w1[:, 385:513].T, e_w1[:, 513:641].T, e_b1[None, :],
        wlabel[None, :], wsec[None, :])

    adjp = jnp.pad(nor_adj_tensor, ((0, PAD), (0, 0)))
    outv = _mlp_call(g3.reshape(SP, D), adjp, m, c1,
                     e_w1[:, 384:385].T, e_w2.T, e_b2[None, :],
                     e_w3.T, e_b3[None, :])

    score2 = _select_call(outv.reshape(NROW, 128))
    scale = jnp.asarray(budget, jnp.float32) / jnp.float32(128)
    score = score2.reshape(SP)[:S] * scale
    return add_feat2.reshape(D), score, sub_graph_nodes.reshape(1, S)
